# baseline (speedup 1.0000x reference)
import jax
import jax.numpy as jnp
from jax.experimental import pallas as pl

_BB = 256


def _body(x_ref, pos_ref, o_ref):
    n, l = x_ref.shape
    d = pos_ref.shape[0]
    y = x_ref[...][:, None, :] + pos_ref[...][None, :, :]
    o_ref[...] = y.reshape(n * d, l)


def kernel(x, pos_table):
    B, L = x.shape
    D = pos_table.shape[-1]
    posT = pos_table.T
    y = pl.pallas_call(
        _body,
        grid=(B // _BB,),
        in_specs=[
            pl.BlockSpec((_BB, L), lambda i: (i, 0)),
            pl.BlockSpec((D, L), lambda i: (0, 0)),
        ],
        out_specs=pl.BlockSpec((_BB * D, L), lambda i: (i, 0)),
        out_shape=jax.ShapeDtypeStruct((B * D, L), x.dtype),
    )(x, posT)
    return y.reshape(B, D, L).transpose(0, 2, 1)
